# baseline (speedup 1.0000x reference)
import functools

import jax
import jax.numpy as jnp
from jax import lax
from jax.experimental import pallas as pl
from jax.experimental.pallas import tpu as pltpu
from jax.experimental.pallas import tpu_sc as plsc

E = 8
D = 1024
H = 2048
T = 2048
BT = 256
NT = T // BT
BM = 256
SLOTS = 2 * T
P = SLOTS + E * BM
NTF = P // BM
NMETA = 48
TS = 512
NS = SLOTS // TS

NW = 32
CHUNK = SLOTS // NW
L = 16


def _pre_body(x_ref, rw_ref, rb_ref, g_ref, b_ref, xn_ref, ew_ref, ww_ref):
    x = x_ref[...]
    mu = jnp.mean(x, axis=1, keepdims=True)
    xc = x - mu
    var = jnp.mean(xc * xc, axis=1, keepdims=True)
    xn = xc * lax.rsqrt(var + 1e-5) * g_ref[0:1, :] + b_ref[0:1, :]
    xn_ref[...] = xn
    lt = lax.dot_general(rw_ref[...], xn, (((1,), (1,)), ((), ())),
                         preferred_element_type=jnp.float32) + rb_ref[:, 0:1]
    rows = lax.broadcasted_iota(jnp.int32, (E, BT), 0)
    m1 = jnp.max(lt, axis=0, keepdims=True)
    i1 = jnp.min(jnp.where(lt == m1, rows, E), axis=0, keepdims=True)
    l2 = jnp.where(rows == i1, -jnp.inf, lt)
    m2 = jnp.max(l2, axis=0, keepdims=True)
    i2 = jnp.min(jnp.where(l2 == m2, rows, E), axis=0, keepdims=True)
    w1 = jax.nn.sigmoid(m1 - m2)
    ew_ref[...] = jnp.where(rows == 0, i1, jnp.where(rows == 1, i2, 0))
    ww_ref[...] = jnp.where(rows == 0, w1, jnp.where(rows == 1, 1.0 - w1, 0.0))


_pre_call = pl.pallas_call(
    _pre_body,
    grid=(NT,),
    in_specs=[
        pl.BlockSpec((BT, D), lambda t: (t, 0)),
        pl.BlockSpec((E, D), lambda t: (0, 0)),
        pl.BlockSpec((E, 8), lambda t: (0, 0)),
        pl.BlockSpec((8, D), lambda t: (0, 0)),
        pl.BlockSpec((8, D), lambda t: (0, 0)),
    ],
    out_specs=[
        pl.BlockSpec((BT, D), lambda t: (t, 0)),
        pl.BlockSpec((E, BT), lambda t: (0, t)),
        pl.BlockSpec((E, BT), lambda t: (0, t)),
    ],
    out_shape=[
        jax.ShapeDtypeStruct((T, D), jnp.float32),
        jax.ShapeDtypeStruct((E, T), jnp.int32),
        jax.ShapeDtypeStruct((E, T), jnp.float32),
    ],
)


def _route_body(ew_ref, pos_ref, meta_ref, rank_scr, acc_scr, tri_scr):
    ph = pl.program_id(0)
    t = pl.program_id(1)
    ev = ew_ref[0]
    erows = lax.broadcasted_iota(jnp.int32, (E, TS), 0)
    oh = jnp.where(ev == erows, 1.0, 0.0)

    @pl.when((ph == 0) & (t == 0))
    def _():
        acc_scr[...] = jnp.zeros_like(acc_scr)
        ir = lax.broadcasted_iota(jnp.int32, (TS, TS), 0)
        ic = lax.broadcasted_iota(jnp.int32, (TS, TS), 1)
        tri_scr[...] = jnp.where(ir < ic, 1.0, 0.0)

    @pl.when(ph == 0)
    def _():
        run = acc_scr[:, 0:1]
        within = lax.dot_general(oh, tri_scr[...], (((1,), (0,)), ((), ())),
                                 preferred_element_type=jnp.float32)
        rank = jnp.sum(oh * (within + run), axis=0, keepdims=True)
        rank_scr[pl.ds(t, 1), :] = rank
        newrun = run + jnp.sum(oh, axis=1, keepdims=True)
        acc_scr[:, 0:1] = newrun

        @pl.when(t == NS - 1)
        def _():
            seg = jnp.floor((newrun + (BM - 1)) / BM) * BM
            er = lax.broadcasted_iota(jnp.int32, (E, E), 0)
            ec = lax.broadcasted_iota(jnp.int32, (E, E), 1)
            tri8 = jnp.where(ec < er, 1.0, 0.0)
            off = lax.dot_general(tri8, seg, (((1,), (0,)), ((), ())),
                                  preferred_element_type=jnp.float32)
            acc_scr[:, 1:2] = off
            acc_scr[:, 2:3] = jnp.broadcast_to(
                jnp.sum(seg, axis=0, keepdims=True), (E, 1))

    @pl.when(ph == 1)
    def _():
        off = acc_scr[:, 1:2]
        off_sel = jnp.sum(oh * off, axis=0, keepdims=True)
        pos = off_sel + rank_scr[pl.ds(t, 1), :]
        pos_ref[0] = pos.astype(jnp.int32)

        @pl.when(t == 0)
        def _():
            ivec = lax.broadcasted_iota(jnp.int32, (1, 128), 1)
            rowe = lax.broadcasted_iota(jnp.int32, (E, 128), 0)
            cmp = jnp.where((rowe >= 1)
                            & (ivec.astype(jnp.float32) * BM >= off),
                            1.0, 0.0)
            te = jnp.sum(cmp, axis=0, keepdims=True)
            used = acc_scr[0:1, 2:3] / BM
            te = jnp.where(ivec >= NTF, used, te)
            meta_ref[...] = jnp.broadcast_to(te.astype(jnp.int32), (8, 128))


_route_call = pl.pallas_call(
    _route_body,
    grid=(2, NS),
    in_specs=[pl.BlockSpec((1, 1, TS), lambda ph, t: (t, 0, 0))],
    out_specs=[
        pl.BlockSpec((1, 1, TS), lambda ph, t: (t, 0, 0)),
        pl.BlockSpec((8, 128), lambda ph, t: (0, 0)),
    ],
    out_shape=[
        jax.ShapeDtypeStruct((NS, 1, TS), jnp.int32),
        jax.ShapeDtypeStruct((8, 128), jnp.int32),
    ],
    scratch_shapes=[
        pltpu.VMEM((NS, TS), jnp.float32),
        pltpu.VMEM((E, 128), jnp.float32),
        pltpu.VMEM((TS, TS), jnp.float32),
    ],
    compiler_params=pltpu.CompilerParams(
        dimension_semantics=("arbitrary", "arbitrary")),
)


_sc_mesh = plsc.VectorSubcoreMesh(core_axis_name="c", subcore_axis_name="s")


@functools.partial(
    pl.kernel,
    out_type=jax.ShapeDtypeStruct((P, D), jnp.float32),
    mesh=_sc_mesh,
    scratch_types=[
        pltpu.VMEM((CHUNK,), jnp.int32),
        pltpu.VMEM((L, D), jnp.float32),
        pltpu.VMEM((L, D), jnp.float32),
        pltpu.SemaphoreType.DMA,
        pltpu.SemaphoreType.DMA,
    ],
)
def _disp(xn_hbm, pos_hbm, xs_hbm, pos_vm, xb0, xb1, sem0, sem1):
    wid = lax.axis_index("s") * 2 + lax.axis_index("c")
    base = wid * CHUNK
    pltpu.sync_copy(pos_hbm.at[pl.ds(base, CHUNK)], pos_vm)
    bufs = (xb0, xb1)
    sems = (sem0, sem1)
    cps = [None, None]
    for k2 in range(CHUNK // L):
        b = k2 % 2
        if cps[b] is not None:
            cps[b].wait()
        tok0 = lax.rem(base + k2 * L, T)
        pltpu.sync_copy(xn_hbm.at[pl.ds(tok0, L)], bufs[b])
        pv = pos_vm[pl.ds(k2 * L, L)]
        cps[b] = pltpu.async_copy(bufs[b], xs_hbm.at[pv], sems[b])
    for cp in cps:
        cp.wait()


def _ffn_body(meta_ref, xs_ref, w1_ref, b1_ref, w2_ref, b2_ref, ys_ref):
    i = pl.program_id(0)

    @pl.when(i < meta_ref[NTF])
    def _():
        e = meta_ref[i]
        xv = xs_ref[...].astype(jnp.bfloat16)
        h = lax.dot_general(xv, w1_ref[0].astype(jnp.bfloat16),
                            (((1,), (1,)), ((), ())),
                            preferred_element_type=jnp.float32)
        h = jnp.maximum(h + b1_ref[pl.ds(e, 1), :], 0.0)
        y = lax.dot_general(h.astype(jnp.bfloat16),
                            w2_ref[0].astype(jnp.bfloat16),
                            (((1,), (1,)), ((), ())),
                            preferred_element_type=jnp.float32)
        ys_ref[...] = y + b2_ref[pl.ds(e, 1), :]


_ffn_call = pl.pallas_call(
    _ffn_body,
    grid_spec=pltpu.PrefetchScalarGridSpec(
        num_scalar_prefetch=1,
        grid=(NTF,),
        in_specs=[
            pl.BlockSpec((BM, D), lambda i, m: (i, 0)),
            pl.BlockSpec((1, H, D), lambda i, m: (m[i], 0, 0)),
            pl.BlockSpec((E, H), lambda i, m: (0, 0)),
            pl.BlockSpec((1, D, H), lambda i, m: (m[i], 0, 0)),
            pl.BlockSpec((E, D), lambda i, m: (0, 0)),
        ],
        out_specs=pl.BlockSpec((BM, D), lambda i, m: (i, 0)),
    ),
    out_shape=jax.ShapeDtypeStruct((P, D), jnp.float32),
    compiler_params=pltpu.CompilerParams(
        dimension_semantics=("arbitrary",)),
)


@functools.partial(
    pl.kernel,
    out_type=[
        jax.ShapeDtypeStruct((T, D), jnp.float32),
        jax.ShapeDtypeStruct((T, D), jnp.float32),
    ],
    mesh=_sc_mesh,
    scratch_types=[
        pltpu.VMEM((T // NW,), jnp.int32),
        pltpu.VMEM((T // NW,), jnp.int32),
        pltpu.VMEM((L, D), jnp.float32),
        pltpu.VMEM((L, D), jnp.float32),
        pltpu.VMEM((L, D), jnp.float32),
        pltpu.VMEM((L, D), jnp.float32),
        pltpu.SemaphoreType.DMA,
        pltpu.SemaphoreType.DMA,
    ],
)
def _gat(pos_hbm, ys_hbm, g0_hbm, g1_hbm,
         p0, p1, ya0, yb0, ya1, yb1, sem0, sem1):
    wid = lax.axis_index("s") * 2 + lax.axis_index("c")
    tpw = T // NW
    t0 = wid * tpw
    pltpu.sync_copy(pos_hbm.at[pl.ds(t0, tpw)], p0)
    pltpu.sync_copy(pos_hbm.at[pl.ds(T + t0, tpw)], p1)
    bufs = ((ya0, yb0), (ya1, yb1))
    sems = (sem0, sem1)
    wcps = [None, None]
    for cc in range(tpw // L):
        b = cc % 2
        y0, y1 = bufs[b]
        if wcps[b] is not None:
            wcps[b][0].wait()
            wcps[b][1].wait()
        tc0 = t0 + cc * L
        i0 = p0[pl.ds(cc * L, L)]
        i1 = p1[pl.ds(cc * L, L)]
        cp0 = pltpu.async_copy(ys_hbm.at[i0], y0, sems[b])
        cp1 = pltpu.async_copy(ys_hbm.at[i1], y1, sems[b])
        cp0.wait()
        cp1.wait()
        wcps[b] = (
            pltpu.async_copy(y0, g0_hbm.at[pl.ds(tc0, L)], sems[b]),
            pltpu.async_copy(y1, g1_hbm.at[pl.ds(tc0, L)], sems[b]),
        )
    for wcp in wcps:
        wcp[0].wait()
        wcp[1].wait()


def _fin_body(x_ref, ww_ref, g0_ref, g1_ref, out_ref):
    sel0 = jnp.where(
        lax.broadcasted_iota(jnp.int32, (E, 8), 0) == 0, 1.0, 0.0)[:, 0:1]
    sel1 = jnp.where(
        lax.broadcasted_iota(jnp.int32, (E, 8), 0) == 1, 1.0, 0.0)[:, 0:1]
    w0c = lax.dot_general(ww_ref[...], sel0, (((0,), (0,)), ((), ())),
                          preferred_element_type=jnp.float32)
    w1c = lax.dot_general(ww_ref[...], sel1, (((0,), (0,)), ((), ())),
                          preferred_element_type=jnp.float32)
    out_ref[...] = (x_ref[...] + w0c * g0_ref[...] + w1c * g1_ref[...])


_fin_call = pl.pallas_call(
    _fin_body,
    grid=(NT,),
    in_specs=[
        pl.BlockSpec((BT, D), lambda t: (t, 0)),
        pl.BlockSpec((E, BT), lambda t: (0, t)),
        pl.BlockSpec((BT, D), lambda t: (t, 0)),
        pl.BlockSpec((BT, D), lambda t: (t, 0)),
    ],
    out_specs=pl.BlockSpec((BT, D), lambda t: (t, 0)),
    out_shape=jax.ShapeDtypeStruct((T, D), jnp.float32),
)


def kernel(x, router_W, router_b, W1, b1, W2, b2, ln_g, ln_b):
    rb2 = jnp.broadcast_to(router_b[:, None], (E, 8))
    g2 = jnp.broadcast_to(ln_g[None, :], (8, D))
    lb2 = jnp.broadcast_to(ln_b[None, :], (8, D))
    xn, ew, ww = _pre_call(x, router_W, rb2, g2, lb2)
    ews = jnp.reshape(jnp.concatenate([ew[0], ew[1]]), (NS, 1, TS))
    pos3, meta8 = _route_call(ews)
    pos = jnp.reshape(pos3, (SLOTS,))
    meta = meta8[0, :NMETA]
    xs = _disp(xn, pos)
    ys = _ffn_call(meta, xs, W1, b1, W2, b2)
    g0, g1 = _gat(pos, ys)
    return _fin_call(x, ww, g0, g1)


# speedup vs baseline: 1.0290x; 1.0290x over previous
import functools

import jax
import jax.numpy as jnp
from jax import lax
from jax.experimental import pallas as pl
from jax.experimental.pallas import tpu as pltpu
from jax.experimental.pallas import tpu_sc as plsc

E = 8
D = 1024
H = 2048
T = 2048
BT = 256
NT = T // BT
BM = 256
SLOTS = 2 * T
P = SLOTS + E * BM
NTF = P // BM
NMETA = 48
TS = 512
NS = SLOTS // TS

NW = 32
CHUNK = SLOTS // NW
L = 16


def _prert_body(x_ref, rw_ref, rb_ref, g_ref, b_ref,
                xn_ref, ww_ref, pos_ref, meta_ref,
                e_scr, rank_scr, acc_scr, tri_scr):
    i = pl.program_id(0)

    @pl.when(i == 0)
    def _():
        acc_scr[...] = jnp.zeros_like(acc_scr)
        ir = lax.broadcasted_iota(jnp.int32, (TS, TS), 0)
        ic = lax.broadcasted_iota(jnp.int32, (TS, TS), 1)
        tri_scr[...] = jnp.where(ir < ic, 1.0, 0.0)

    @pl.when(i < NT)
    def _():
        x = x_ref[...]
        mu = jnp.mean(x, axis=1, keepdims=True)
        xc = x - mu
        var = jnp.mean(xc * xc, axis=1, keepdims=True)
        xn = xc * lax.rsqrt(var + 1e-5) * g_ref[0:1, :] + b_ref[0:1, :]
        xn_ref[...] = xn
        lt = lax.dot_general(rw_ref[...], xn, (((1,), (1,)), ((), ())),
                             preferred_element_type=jnp.float32) + rb_ref[:, 0:1]
        rows = lax.broadcasted_iota(jnp.int32, (E, BT), 0)
        m1 = jnp.max(lt, axis=0, keepdims=True)
        i1 = jnp.min(jnp.where(lt == m1, rows, E), axis=0, keepdims=True)
        l2 = jnp.where(rows == i1, -jnp.inf, lt)
        m2 = jnp.max(l2, axis=0, keepdims=True)
        i2 = jnp.min(jnp.where(l2 == m2, rows, E), axis=0, keepdims=True)
        w1 = jax.nn.sigmoid(m1 - m2)
        ww_ref[...] = jnp.where(rows == 0, w1,
                                jnp.where(rows == 1, 1.0 - w1, 0.0))
        col = pl.multiple_of((lax.rem(i, 2)) * BT, BT)
        r0 = i // 2
        e_scr[pl.ds(r0, 1), pl.ds(col, BT)] = i1
        e_scr[pl.ds(NS // 2 + r0, 1), pl.ds(col, BT)] = i2

    @pl.when((i >= NT) & (i < NT + NS))
    def _():
        t = i - NT
        ev = e_scr[pl.ds(t, 1), :]
        erows = lax.broadcasted_iota(jnp.int32, (E, TS), 0)
        oh = jnp.where(ev == erows, 1.0, 0.0)
        run = acc_scr[:, 0:1]
        within = lax.dot_general(oh, tri_scr[...], (((1,), (0,)), ((), ())),
                                 preferred_element_type=jnp.float32)
        rank = jnp.sum(oh * (within + run), axis=0, keepdims=True)
        rank_scr[pl.ds(t, 1), :] = rank
        newrun = run + jnp.sum(oh, axis=1, keepdims=True)
        acc_scr[:, 0:1] = newrun

        @pl.when(t == NS - 1)
        def _():
            seg = jnp.floor((newrun + (BM - 1)) / BM) * BM
            er = lax.broadcasted_iota(jnp.int32, (E, E), 0)
            ec = lax.broadcasted_iota(jnp.int32, (E, E), 1)
            tri8 = jnp.where(ec < er, 1.0, 0.0)
            off = lax.dot_general(tri8, seg, (((1,), (0,)), ((), ())),
                                  preferred_element_type=jnp.float32)
            acc_scr[:, 1:2] = off
            acc_scr[:, 2:3] = jnp.broadcast_to(
                jnp.sum(seg, axis=0, keepdims=True), (E, 1))

    @pl.when(i >= NT + NS)
    def _():
        t = i - (NT + NS)
        ev = e_scr[pl.ds(t, 1), :]
        erows = lax.broadcasted_iota(jnp.int32, (E, TS), 0)
        oh = jnp.where(ev == erows, 1.0, 0.0)
        off = acc_scr[:, 1:2]
        off_sel = jnp.sum(oh * off, axis=0, keepdims=True)
        pos = off_sel + rank_scr[pl.ds(t, 1), :]
        pos_ref[0] = pos.astype(jnp.int32)

        @pl.when(t == 0)
        def _():
            ivec = lax.broadcasted_iota(jnp.int32, (1, 128), 1)
            rowe = lax.broadcasted_iota(jnp.int32, (E, 128), 0)
            cmp = jnp.where((rowe >= 1)
                            & (ivec.astype(jnp.float32) * BM >= off),
                            1.0, 0.0)
            te = jnp.sum(cmp, axis=0, keepdims=True)
            used = acc_scr[0:1, 2:3] / BM
            te = jnp.where(ivec >= NTF, used, te)
            meta_ref[...] = jnp.broadcast_to(te.astype(jnp.int32), (8, 128))


def _clip7(i):
    return jnp.minimum(i, NT - 1)


_prert_call = pl.pallas_call(
    _prert_body,
    grid=(NT + 2 * NS,),
    in_specs=[
        pl.BlockSpec((BT, D), lambda i: (_clip7(i), 0)),
        pl.BlockSpec((E, D), lambda i: (0, 0)),
        pl.BlockSpec((E, 8), lambda i: (0, 0)),
        pl.BlockSpec((8, D), lambda i: (0, 0)),
        pl.BlockSpec((8, D), lambda i: (0, 0)),
    ],
    out_specs=[
        pl.BlockSpec((BT, D), lambda i: (_clip7(i), 0)),
        pl.BlockSpec((E, BT), lambda i: (0, _clip7(i))),
        pl.BlockSpec((1, 1, TS),
                     lambda i: (jnp.clip(i - (NT + NS), 0, NS - 1), 0, 0)),
        pl.BlockSpec((8, 128), lambda i: (0, 0)),
    ],
    out_shape=[
        jax.ShapeDtypeStruct((T, D), jnp.float32),
        jax.ShapeDtypeStruct((E, T), jnp.float32),
        jax.ShapeDtypeStruct((NS, 1, TS), jnp.int32),
        jax.ShapeDtypeStruct((8, 128), jnp.int32),
    ],
    scratch_shapes=[
        pltpu.VMEM((NS, TS), jnp.int32),
        pltpu.VMEM((NS, TS), jnp.float32),
        pltpu.VMEM((E, 128), jnp.float32),
        pltpu.VMEM((TS, TS), jnp.float32),
    ],
    compiler_params=pltpu.CompilerParams(
        dimension_semantics=("arbitrary",)),
)


_sc_mesh = plsc.VectorSubcoreMesh(core_axis_name="c", subcore_axis_name="s")


@functools.partial(
    pl.kernel,
    out_type=jax.ShapeDtypeStruct((P, D), jnp.float32),
    mesh=_sc_mesh,
    scratch_types=[
        pltpu.VMEM((CHUNK,), jnp.int32),
        pltpu.VMEM((L, D), jnp.float32),
        pltpu.VMEM((L, D), jnp.float32),
        pltpu.SemaphoreType.DMA,
        pltpu.SemaphoreType.DMA,
    ],
)
def _disp(xn_hbm, pos_hbm, xs_hbm, pos_vm, xb0, xb1, sem0, sem1):
    wid = lax.axis_index("s") * 2 + lax.axis_index("c")
    base = wid * CHUNK
    pltpu.sync_copy(pos_hbm.at[pl.ds(base, CHUNK)], pos_vm)
    bufs = (xb0, xb1)
    sems = (sem0, sem1)
    cps = [None, None]
    for k2 in range(CHUNK // L):
        b = k2 % 2
        if cps[b] is not None:
            cps[b].wait()
        tok0 = lax.rem(base + k2 * L, T)
        pltpu.sync_copy(xn_hbm.at[pl.ds(tok0, L)], bufs[b])
        pv = pos_vm[pl.ds(k2 * L, L)]
        cps[b] = pltpu.async_copy(bufs[b], xs_hbm.at[pv], sems[b])
    for cp in cps:
        cp.wait()


def _ffn_body(meta_ref, xs_ref, w1_ref, b1_ref, w2_ref, b2_ref, ys_ref):
    i = pl.program_id(0)

    @pl.when(i < meta_ref[NTF])
    def _():
        e = meta_ref[i]
        xv = xs_ref[...].astype(jnp.bfloat16)
        h = lax.dot_general(xv, w1_ref[0].astype(jnp.bfloat16),
                            (((1,), (1,)), ((), ())),
                            preferred_element_type=jnp.float32)
        h = jnp.maximum(h + b1_ref[pl.ds(e, 1), :], 0.0)
        y = lax.dot_general(h.astype(jnp.bfloat16),
                            w2_ref[0].astype(jnp.bfloat16),
                            (((1,), (1,)), ((), ())),
                            preferred_element_type=jnp.float32)
        ys_ref[...] = y + b2_ref[pl.ds(e, 1), :]


_ffn_call = pl.pallas_call(
    _ffn_body,
    grid_spec=pltpu.PrefetchScalarGridSpec(
        num_scalar_prefetch=1,
        grid=(NTF,),
        in_specs=[
            pl.BlockSpec((BM, D), lambda i, m: (i, 0)),
            pl.BlockSpec((1, H, D), lambda i, m: (m[i], 0, 0)),
            pl.BlockSpec((E, H), lambda i, m: (0, 0)),
            pl.BlockSpec((1, D, H), lambda i, m: (m[i], 0, 0)),
            pl.BlockSpec((E, D), lambda i, m: (0, 0)),
        ],
        out_specs=pl.BlockSpec((BM, D), lambda i, m: (i, 0)),
    ),
    out_shape=jax.ShapeDtypeStruct((P, D), jnp.float32),
    compiler_params=pltpu.CompilerParams(
        dimension_semantics=("arbitrary",)),
)


@functools.partial(
    pl.kernel,
    out_type=[
        jax.ShapeDtypeStruct((T, D), jnp.float32),
        jax.ShapeDtypeStruct((T, D), jnp.float32),
    ],
    mesh=_sc_mesh,
    scratch_types=[
        pltpu.VMEM((T // NW,), jnp.int32),
        pltpu.VMEM((T // NW,), jnp.int32),
        pltpu.VMEM((L, D), jnp.float32),
        pltpu.VMEM((L, D), jnp.float32),
        pltpu.VMEM((L, D), jnp.float32),
        pltpu.VMEM((L, D), jnp.float32),
        pltpu.SemaphoreType.DMA,
        pltpu.SemaphoreType.DMA,
    ],
)
def _gat(pos_hbm, ys_hbm, g0_hbm, g1_hbm,
         p0, p1, ya0, yb0, ya1, yb1, sem0, sem1):
    wid = lax.axis_index("s") * 2 + lax.axis_index("c")
    tpw = T // NW
    t0 = wid * tpw
    pltpu.sync_copy(pos_hbm.at[pl.ds(t0, tpw)], p0)
    pltpu.sync_copy(pos_hbm.at[pl.ds(T + t0, tpw)], p1)
    bufs = ((ya0, yb0), (ya1, yb1))
    sems = (sem0, sem1)
    wcps = [None, None]
    for cc in range(tpw // L):
        b = cc % 2
        y0, y1 = bufs[b]
        if wcps[b] is not None:
            wcps[b][0].wait()
            wcps[b][1].wait()
        tc0 = t0 + cc * L
        i0 = p0[pl.ds(cc * L, L)]
        i1 = p1[pl.ds(cc * L, L)]
        cp0 = pltpu.async_copy(ys_hbm.at[i0], y0, sems[b])
        cp1 = pltpu.async_copy(ys_hbm.at[i1], y1, sems[b])
        cp0.wait()
        cp1.wait()
        wcps[b] = (
            pltpu.async_copy(y0, g0_hbm.at[pl.ds(tc0, L)], sems[b]),
            pltpu.async_copy(y1, g1_hbm.at[pl.ds(tc0, L)], sems[b]),
        )
    for wcp in wcps:
        wcp[0].wait()
        wcp[1].wait()


def _fin_body(x_ref, ww_ref, g0_ref, g1_ref, out_ref):
    sel0 = jnp.where(
        lax.broadcasted_iota(jnp.int32, (E, 8), 0) == 0, 1.0, 0.0)[:, 0:1]
    sel1 = jnp.where(
        lax.broadcasted_iota(jnp.int32, (E, 8), 0) == 1, 1.0, 0.0)[:, 0:1]
    w0c = lax.dot_general(ww_ref[...], sel0, (((0,), (0,)), ((), ())),
                          preferred_element_type=jnp.float32)
    w1c = lax.dot_general(ww_ref[...], sel1, (((0,), (0,)), ((), ())),
                          preferred_element_type=jnp.float32)
    out_ref[...] = (x_ref[...] + w0c * g0_ref[...] + w1c * g1_ref[...])


_fin_call = pl.pallas_call(
    _fin_body,
    grid=(NT,),
    in_specs=[
        pl.BlockSpec((BT, D), lambda t: (t, 0)),
        pl.BlockSpec((E, BT), lambda t: (0, t)),
        pl.BlockSpec((BT, D), lambda t: (t, 0)),
        pl.BlockSpec((BT, D), lambda t: (t, 0)),
    ],
    out_specs=pl.BlockSpec((BT, D), lambda t: (t, 0)),
    out_shape=jax.ShapeDtypeStruct((T, D), jnp.float32),
)


def kernel(x, router_W, router_b, W1, b1, W2, b2, ln_g, ln_b):
    rb2 = jnp.broadcast_to(router_b[:, None], (E, 8))
    g2 = jnp.broadcast_to(ln_g[None, :], (8, D))
    lb2 = jnp.broadcast_to(ln_b[None, :], (8, D))
    xn, ww, pos3, meta8 = _prert_call(x, router_W, rb2, g2, lb2)
    pos = jnp.reshape(pos3, (SLOTS,))
    meta = meta8[0, :NMETA]
    xs = _disp(xn, pos)
    ys = _ffn_call(meta, xs, W1, b1, W2, b2)
    g0, g1 = _gat(pos, ys)
    return _fin_call(x, ww, g0, g1)


# speedup vs baseline: 1.1320x; 1.1001x over previous
import functools

import jax
import jax.numpy as jnp
from jax import lax
from jax.experimental import pallas as pl
from jax.experimental.pallas import tpu as pltpu
from jax.experimental.pallas import tpu_sc as plsc

E = 8
D = 1024
H = 2048
T = 2048
BT = 256
NT = T // BT
BM = 256
SLOTS = 2 * T
P = SLOTS + E * BM
NTF = P // BM
NMETA = 48
TS = 512
NS = SLOTS // TS

NW = 32
CHUNK = SLOTS // NW
L = 16
DP = D // 2


def _pack_bf16(x):
    u = lax.bitcast_convert_type(x, jnp.uint32)
    b = (u + jnp.uint32(0x7FFF) + ((u >> 16) & jnp.uint32(1))) >> 16
    hi, lo = b[:, :DP], b[:, DP:]
    return lax.bitcast_convert_type((hi << 16) | lo, jnp.int32)


def _unpack_bf16(p):
    pu = lax.bitcast_convert_type(p, jnp.uint32)
    first = lax.bitcast_convert_type(pu & jnp.uint32(0xFFFF0000), jnp.float32)
    second = lax.bitcast_convert_type(pu << 16, jnp.float32)
    return jnp.concatenate([first, second], axis=1)


def _prert_body(x_ref, rw_ref, rb_ref, g_ref, b_ref,
                xn_ref, ww_ref, pos_ref, meta_ref,
                e_scr, rank_scr, acc_scr, tri_scr):
    i = pl.program_id(0)

    @pl.when(i == 0)
    def _():
        acc_scr[...] = jnp.zeros_like(acc_scr)
        ir = lax.broadcasted_iota(jnp.int32, (TS, TS), 0)
        ic = lax.broadcasted_iota(jnp.int32, (TS, TS), 1)
        tri_scr[...] = jnp.where(ir < ic, 1.0, 0.0)

    @pl.when(i < NT)
    def _():
        x = x_ref[...]
        mu = jnp.mean(x, axis=1, keepdims=True)
        xc = x - mu
        var = jnp.mean(xc * xc, axis=1, keepdims=True)
        xn = xc * lax.rsqrt(var + 1e-5) * g_ref[0:1, :] + b_ref[0:1, :]
        xn_ref[...] = _pack_bf16(xn)
        lt = lax.dot_general(rw_ref[...], xn, (((1,), (1,)), ((), ())),
                             preferred_element_type=jnp.float32) + rb_ref[:, 0:1]
        rows = lax.broadcasted_iota(jnp.int32, (E, BT), 0)
        m1 = jnp.max(lt, axis=0, keepdims=True)
        i1 = jnp.min(jnp.where(lt == m1, rows, E), axis=0, keepdims=True)
        l2 = jnp.where(rows == i1, -jnp.inf, lt)
        m2 = jnp.max(l2, axis=0, keepdims=True)
        i2 = jnp.min(jnp.where(l2 == m2, rows, E), axis=0, keepdims=True)
        w1 = jax.nn.sigmoid(m1 - m2)
        ww_ref[...] = jnp.where(rows == 0, w1,
                                jnp.where(rows == 1, 1.0 - w1, 0.0))
        col = pl.multiple_of((lax.rem(i, 2)) * BT, BT)
        r0 = i // 2
        e_scr[pl.ds(r0, 1), pl.ds(col, BT)] = i1
        e_scr[pl.ds(NS // 2 + r0, 1), pl.ds(col, BT)] = i2

    @pl.when((i >= NT) & (i < NT + NS))
    def _():
        t = i - NT
        ev = e_scr[pl.ds(t, 1), :]
        erows = lax.broadcasted_iota(jnp.int32, (E, TS), 0)
        oh = jnp.where(ev == erows, 1.0, 0.0)
        run = acc_scr[:, 0:1]
        within = lax.dot_general(oh, tri_scr[...], (((1,), (0,)), ((), ())),
                                 preferred_element_type=jnp.float32)
        rank = jnp.sum(oh * (within + run), axis=0, keepdims=True)
        rank_scr[pl.ds(t, 1), :] = rank
        newrun = run + jnp.sum(oh, axis=1, keepdims=True)
        acc_scr[:, 0:1] = newrun

        @pl.when(t == NS - 1)
        def _():
            seg = jnp.floor((newrun + (BM - 1)) / BM) * BM
            er = lax.broadcasted_iota(jnp.int32, (E, E), 0)
            ec = lax.broadcasted_iota(jnp.int32, (E, E), 1)
            tri8 = jnp.where(ec < er, 1.0, 0.0)
            off = lax.dot_general(tri8, seg, (((1,), (0,)), ((), ())),
                                  preferred_element_type=jnp.float32)
            acc_scr[:, 1:2] = off
            acc_scr[:, 2:3] = jnp.broadcast_to(
                jnp.sum(seg, axis=0, keepdims=True), (E, 1))

    @pl.when(i >= NT + NS)
    def _():
        t = i - (NT + NS)
        ev = e_scr[pl.ds(t, 1), :]
        erows = lax.broadcasted_iota(jnp.int32, (E, TS), 0)
        oh = jnp.where(ev == erows, 1.0, 0.0)
        off = acc_scr[:, 1:2]
        off_sel = jnp.sum(oh * off, axis=0, keepdims=True)
        pos = off_sel + rank_scr[pl.ds(t, 1), :]
        pos_ref[0] = pos.astype(jnp.int32)

        @pl.when(t == 0)
        def _():
            ivec = lax.broadcasted_iota(jnp.int32, (1, 128), 1)
            rowe = lax.broadcasted_iota(jnp.int32, (E, 128), 0)
            cmp = jnp.where((rowe >= 1)
                            & (ivec.astype(jnp.float32) * BM >= off),
                            1.0, 0.0)
            te = jnp.sum(cmp, axis=0, keepdims=True)
            used = acc_scr[0:1, 2:3] / BM
            te = jnp.where(ivec >= NTF, used, te)
            meta_ref[...] = jnp.broadcast_to(te.astype(jnp.int32), (8, 128))


def _clip7(i):
    return jnp.minimum(i, NT - 1)


_prert_call = pl.pallas_call(
    _prert_body,
    grid=(NT + 2 * NS,),
    in_specs=[
        pl.BlockSpec((BT, D), lambda i: (_clip7(i), 0)),
        pl.BlockSpec((E, D), lambda i: (0, 0)),
        pl.BlockSpec((E, 8), lambda i: (0, 0)),
        pl.BlockSpec((8, D), lambda i: (0, 0)),
        pl.BlockSpec((8, D), lambda i: (0, 0)),
    ],
    out_specs=[
        pl.BlockSpec((BT, DP), lambda i: (_clip7(i), 0)),
        pl.BlockSpec((E, BT), lambda i: (0, _clip7(i))),
        pl.BlockSpec((1, 1, TS),
                     lambda i: (jnp.clip(i - (NT + NS), 0, NS - 1), 0, 0)),
        pl.BlockSpec((8, 128), lambda i: (0, 0)),
    ],
    out_shape=[
        jax.ShapeDtypeStruct((T, DP), jnp.int32),
        jax.ShapeDtypeStruct((E, T), jnp.float32),
        jax.ShapeDtypeStruct((NS, 1, TS), jnp.int32),
        jax.ShapeDtypeStruct((8, 128), jnp.int32),
    ],
    scratch_shapes=[
        pltpu.VMEM((NS, TS), jnp.int32),
        pltpu.VMEM((NS, TS), jnp.float32),
        pltpu.VMEM((E, 128), jnp.float32),
        pltpu.VMEM((TS, TS), jnp.float32),
    ],
    compiler_params=pltpu.CompilerParams(
        dimension_semantics=("arbitrary",)),
)


_sc_mesh = plsc.VectorSubcoreMesh(core_axis_name="c", subcore_axis_name="s")


@functools.partial(
    pl.kernel,
    out_type=jax.ShapeDtypeStruct((P, DP), jnp.int32),
    mesh=_sc_mesh,
    scratch_types=[
        pltpu.VMEM((CHUNK,), jnp.int32),
        pltpu.VMEM((L, DP), jnp.int32),
        pltpu.VMEM((L, DP), jnp.int32),
        pltpu.SemaphoreType.DMA,
        pltpu.SemaphoreType.DMA,
    ],
)
def _disp(xn_hbm, pos_hbm, xs_hbm, pos_vm, xb0, xb1, sem0, sem1):
    wid = lax.axis_index("s") * 2 + lax.axis_index("c")
    base = wid * CHUNK
    pltpu.sync_copy(pos_hbm.at[pl.ds(base, CHUNK)], pos_vm)
    bufs = (xb0, xb1)
    sems = (sem0, sem1)
    cps = [None, None]
    for k2 in range(CHUNK // L):
        b = k2 % 2
        if cps[b] is not None:
            cps[b].wait()
        tok0 = lax.rem(base + k2 * L, T)
        pltpu.sync_copy(xn_hbm.at[pl.ds(tok0, L)], bufs[b])
        pv = pos_vm[pl.ds(k2 * L, L)]
        cps[b] = pltpu.async_copy(bufs[b], xs_hbm.at[pv], sems[b])
    for cp in cps:
        cp.wait()


def _ffn_body(meta_ref, xs_ref, w1_ref, b1_ref, w2_ref, b2_ref, ys_ref):
    i = pl.program_id(0)

    @pl.when(i < meta_ref[NTF])
    def _():
        e = meta_ref[i]
        xv = _unpack_bf16(xs_ref[...]).astype(jnp.bfloat16)
        h = lax.dot_general(xv, w1_ref[0].astype(jnp.bfloat16),
                            (((1,), (1,)), ((), ())),
                            preferred_element_type=jnp.float32)
        h = jnp.maximum(h + b1_ref[pl.ds(e, 1), :], 0.0)
        y = lax.dot_general(h.astype(jnp.bfloat16),
                            w2_ref[0].astype(jnp.bfloat16),
                            (((1,), (1,)), ((), ())),
                            preferred_element_type=jnp.float32)
        ys_ref[...] = _pack_bf16(y + b2_ref[pl.ds(e, 1), :])


_ffn_call = pl.pallas_call(
    _ffn_body,
    grid_spec=pltpu.PrefetchScalarGridSpec(
        num_scalar_prefetch=1,
        grid=(NTF,),
        in_specs=[
            pl.BlockSpec((BM, DP), lambda i, m: (i, 0)),
            pl.BlockSpec((1, H, D), lambda i, m: (m[i], 0, 0)),
            pl.BlockSpec((E, H), lambda i, m: (0, 0)),
            pl.BlockSpec((1, D, H), lambda i, m: (m[i], 0, 0)),
            pl.BlockSpec((E, D), lambda i, m: (0, 0)),
        ],
        out_specs=pl.BlockSpec((BM, DP), lambda i, m: (i, 0)),
    ),
    out_shape=jax.ShapeDtypeStruct((P, DP), jnp.int32),
    compiler_params=pltpu.CompilerParams(
        dimension_semantics=("arbitrary",)),
)


@functools.partial(
    pl.kernel,
    out_type=[
        jax.ShapeDtypeStruct((T, DP), jnp.int32),
        jax.ShapeDtypeStruct((T, DP), jnp.int32),
    ],
    mesh=_sc_mesh,
    scratch_types=[
        pltpu.VMEM((T // NW,), jnp.int32),
        pltpu.VMEM((T // NW,), jnp.int32),
        pltpu.VMEM((L, DP), jnp.int32),
        pltpu.VMEM((L, DP), jnp.int32),
        pltpu.VMEM((L, DP), jnp.int32),
        pltpu.VMEM((L, DP), jnp.int32),
        pltpu.SemaphoreType.DMA,
        pltpu.SemaphoreType.DMA,
    ],
)
def _gat(pos_hbm, ys_hbm, g0_hbm, g1_hbm,
         p0, p1, ya0, yb0, ya1, yb1, sem0, sem1):
    wid = lax.axis_index("s") * 2 + lax.axis_index("c")
    tpw = T // NW
    t0 = wid * tpw
    pltpu.sync_copy(pos_hbm.at[pl.ds(t0, tpw)], p0)
    pltpu.sync_copy(pos_hbm.at[pl.ds(T + t0, tpw)], p1)
    bufs = ((ya0, yb0), (ya1, yb1))
    sems = (sem0, sem1)
    wcps = [None, None]
    for cc in range(tpw // L):
        b = cc % 2
        y0, y1 = bufs[b]
        if wcps[b] is not None:
            wcps[b][0].wait()
            wcps[b][1].wait()
        tc0 = t0 + cc * L
        i0 = p0[pl.ds(cc * L, L)]
        i1 = p1[pl.ds(cc * L, L)]
        cp0 = pltpu.async_copy(ys_hbm.at[i0], y0, sems[b])
        cp1 = pltpu.async_copy(ys_hbm.at[i1], y1, sems[b])
        cp0.wait()
        cp1.wait()
        wcps[b] = (
            pltpu.async_copy(y0, g0_hbm.at[pl.ds(tc0, L)], sems[b]),
            pltpu.async_copy(y1, g1_hbm.at[pl.ds(tc0, L)], sems[b]),
        )
    for wcp in wcps:
        wcp[0].wait()
        wcp[1].wait()


def _fin_body(x_ref, ww_ref, g0_ref, g1_ref, out_ref):
    sel0 = jnp.where(
        lax.broadcasted_iota(jnp.int32, (E, 8), 0) == 0, 1.0, 0.0)[:, 0:1]
    sel1 = jnp.where(
        lax.broadcasted_iota(jnp.int32, (E, 8), 0) == 1, 1.0, 0.0)[:, 0:1]
    w0c = lax.dot_general(ww_ref[...], sel0, (((0,), (0,)), ((), ())),
                          preferred_element_type=jnp.float32)
    w1c = lax.dot_general(ww_ref[...], sel1, (((0,), (0,)), ((), ())),
                          preferred_element_type=jnp.float32)
    g0 = _unpack_bf16(g0_ref[...])
    g1 = _unpack_bf16(g1_ref[...])
    out_ref[...] = x_ref[...] + w0c * g0 + w1c * g1


_fin_call = pl.pallas_call(
    _fin_body,
    grid=(NT,),
    in_specs=[
        pl.BlockSpec((BT, D), lambda t: (t, 0)),
        pl.BlockSpec((E, BT), lambda t: (0, t)),
        pl.BlockSpec((BT, DP), lambda t: (t, 0)),
        pl.BlockSpec((BT, DP), lambda t: (t, 0)),
    ],
    out_specs=pl.BlockSpec((BT, D), lambda t: (t, 0)),
    out_shape=jax.ShapeDtypeStruct((T, D), jnp.float32),
)


def kernel(x, router_W, router_b, W1, b1, W2, b2, ln_g, ln_b):
    rb2 = jnp.broadcast_to(router_b[:, None], (E, 8))
    g2 = jnp.broadcast_to(ln_g[None, :], (8, D))
    lb2 = jnp.broadcast_to(ln_b[None, :], (8, D))
    xn, ww, pos3, meta8 = _prert_call(x, router_W, rb2, g2, lb2)
    pos = jnp.reshape(pos3, (SLOTS,))
    meta = meta8[0, :NMETA]
    xs = _disp(xn, pos)
    ys = _ffn_call(meta, xs, W1, b1, W2, b2)
    g0, g1 = _gat(pos, ys)
    return _fin_call(x, ww, g0, g1)


# speedup vs baseline: 1.1550x; 1.0203x over previous
import functools

import jax
import jax.numpy as jnp
from jax import lax
from jax.experimental import pallas as pl
from jax.experimental.pallas import tpu as pltpu
from jax.experimental.pallas import tpu_sc as plsc

E = 8
D = 1024
H = 2048
T = 2048
BT = 256
NT = T // BT
BM = 256
SLOTS = 2 * T
P = SLOTS + E * BM
NTF = P // BM
NMETA = 48
TS = 512
NS = SLOTS // TS

NW = 32
CHUNK = SLOTS // NW
L = 16
DP = D // 2


def _pack_bf16(x):
    u = lax.bitcast_convert_type(x, jnp.uint32)
    b = (u + jnp.uint32(0x7FFF) + ((u >> 16) & jnp.uint32(1))) >> 16
    hi, lo = b[:, :DP], b[:, DP:]
    return lax.bitcast_convert_type((hi << 16) | lo, jnp.int32)


def _unpack_bf16(p):
    pu = lax.bitcast_convert_type(p, jnp.uint32)
    first = lax.bitcast_convert_type(pu & jnp.uint32(0xFFFF0000), jnp.float32)
    second = lax.bitcast_convert_type(pu << 16, jnp.float32)
    return jnp.concatenate([first, second], axis=1)


def _prert_body(x_ref, rw_ref, rb_ref, g_ref, b_ref,
                xn_ref, ww_ref, pos_ref, meta_ref,
                e_scr, rank_scr, acc_scr, tri_scr):
    i = pl.program_id(0)

    @pl.when(i == 0)
    def _():
        acc_scr[...] = jnp.zeros_like(acc_scr)
        ir = lax.broadcasted_iota(jnp.int32, (TS, TS), 0)
        ic = lax.broadcasted_iota(jnp.int32, (TS, TS), 1)
        tri_scr[...] = jnp.where(ir < ic, 1.0, 0.0)

    @pl.when(i < NT)
    def _():
        x = x_ref[...]
        mu = jnp.mean(x, axis=1, keepdims=True)
        xc = x - mu
        var = jnp.mean(xc * xc, axis=1, keepdims=True)
        xn = xc * lax.rsqrt(var + 1e-5) * g_ref[0:1, :] + b_ref[0:1, :]
        xn_ref[...] = _pack_bf16(xn)
        lt = lax.dot_general(rw_ref[...], xn, (((1,), (1,)), ((), ())),
                             preferred_element_type=jnp.float32) + rb_ref[:, 0:1]
        rows = lax.broadcasted_iota(jnp.int32, (E, BT), 0)
        m1 = jnp.max(lt, axis=0, keepdims=True)
        i1 = jnp.min(jnp.where(lt == m1, rows, E), axis=0, keepdims=True)
        l2 = jnp.where(rows == i1, -jnp.inf, lt)
        m2 = jnp.max(l2, axis=0, keepdims=True)
        i2 = jnp.min(jnp.where(l2 == m2, rows, E), axis=0, keepdims=True)
        w1 = jax.nn.sigmoid(m1 - m2)
        ww_ref[...] = jnp.where(rows == 0, w1,
                                jnp.where(rows == 1, 1.0 - w1, 0.0))
        col = pl.multiple_of((lax.rem(i, 2)) * BT, BT)
        r0 = i // 2
        e_scr[pl.ds(r0, 1), pl.ds(col, BT)] = i1
        e_scr[pl.ds(NS // 2 + r0, 1), pl.ds(col, BT)] = i2

    @pl.when((i >= NT) & (i < NT + NS))
    def _():
        t = i - NT
        ev = e_scr[pl.ds(t, 1), :]
        erows = lax.broadcasted_iota(jnp.int32, (E, TS), 0)
        oh = jnp.where(ev == erows, 1.0, 0.0)
        run = acc_scr[:, 0:1]
        within = lax.dot_general(oh, tri_scr[...], (((1,), (0,)), ((), ())),
                                 preferred_element_type=jnp.float32)
        rank = jnp.sum(oh * (within + run), axis=0, keepdims=True)
        rank_scr[pl.ds(t, 1), :] = rank
        newrun = run + jnp.sum(oh, axis=1, keepdims=True)
        acc_scr[:, 0:1] = newrun

        @pl.when(t == NS - 1)
        def _():
            seg = jnp.floor((newrun + (BM - 1)) / BM) * BM
            er = lax.broadcasted_iota(jnp.int32, (E, E), 0)
            ec = lax.broadcasted_iota(jnp.int32, (E, E), 1)
            tri8 = jnp.where(ec < er, 1.0, 0.0)
            off = lax.dot_general(tri8, seg, (((1,), (0,)), ((), ())),
                                  preferred_element_type=jnp.float32)
            acc_scr[:, 1:2] = off
            acc_scr[:, 2:3] = jnp.broadcast_to(
                jnp.sum(seg, axis=0, keepdims=True), (E, 1))

    @pl.when(i >= NT + NS)
    def _():
        t = i - (NT + NS)
        ev = e_scr[pl.ds(t, 1), :]
        erows = lax.broadcasted_iota(jnp.int32, (E, TS), 0)
        oh = jnp.where(ev == erows, 1.0, 0.0)
        off = acc_scr[:, 1:2]
        off_sel = jnp.sum(oh * off, axis=0, keepdims=True)
        pos = off_sel + rank_scr[pl.ds(t, 1), :]
        pos_ref[0] = pos.astype(jnp.int32)

        @pl.when(t == 0)
        def _():
            ivec = lax.broadcasted_iota(jnp.int32, (1, 128), 1)
            rowe = lax.broadcasted_iota(jnp.int32, (E, 128), 0)
            cmp = jnp.where((rowe >= 1)
                            & (ivec.astype(jnp.float32) * BM >= off),
                            1.0, 0.0)
            te = jnp.sum(cmp, axis=0, keepdims=True)
            used = acc_scr[0:1, 2:3] / BM
            te = jnp.where(ivec >= NTF, used, te)
            te = jnp.where(ivec >= 64,
                           jnp.clip(ivec.astype(jnp.float32) - 64.0,
                                    0.0, used - 1.0), te)
            meta_ref[...] = jnp.broadcast_to(te.astype(jnp.int32), (8, 128))


def _clip7(i):
    return jnp.minimum(i, NT - 1)


_prert_call = pl.pallas_call(
    _prert_body,
    grid=(NT + 2 * NS,),
    in_specs=[
        pl.BlockSpec((BT, D), lambda i: (_clip7(i), 0)),
        pl.BlockSpec((E, D), lambda i: (0, 0)),
        pl.BlockSpec((E, 8), lambda i: (0, 0)),
        pl.BlockSpec((8, D), lambda i: (0, 0)),
        pl.BlockSpec((8, D), lambda i: (0, 0)),
    ],
    out_specs=[
        pl.BlockSpec((BT, DP), lambda i: (_clip7(i), 0)),
        pl.BlockSpec((E, BT), lambda i: (0, _clip7(i))),
        pl.BlockSpec((1, 1, TS),
                     lambda i: (jnp.clip(i - (NT + NS), 0, NS - 1), 0, 0)),
        pl.BlockSpec((8, 128), lambda i: (0, 0)),
    ],
    out_shape=[
        jax.ShapeDtypeStruct((T, DP), jnp.int32),
        jax.ShapeDtypeStruct((E, T), jnp.float32),
        jax.ShapeDtypeStruct((NS, 1, TS), jnp.int32),
        jax.ShapeDtypeStruct((8, 128), jnp.int32),
    ],
    scratch_shapes=[
        pltpu.VMEM((NS, TS), jnp.int32),
        pltpu.VMEM((NS, TS), jnp.float32),
        pltpu.VMEM((E, 128), jnp.float32),
        pltpu.VMEM((TS, TS), jnp.float32),
    ],
    compiler_params=pltpu.CompilerParams(
        dimension_semantics=("arbitrary",)),
)


_sc_mesh = plsc.VectorSubcoreMesh(core_axis_name="c", subcore_axis_name="s")


@functools.partial(
    pl.kernel,
    out_type=jax.ShapeDtypeStruct((P, DP), jnp.int32),
    mesh=_sc_mesh,
    scratch_types=[
        pltpu.VMEM((CHUNK,), jnp.int32),
        pltpu.VMEM((L, DP), jnp.int32),
        pltpu.VMEM((L, DP), jnp.int32),
        pltpu.SemaphoreType.DMA,
        pltpu.SemaphoreType.DMA,
    ],
)
def _disp(xn_hbm, pos_hbm, xs_hbm, pos_vm, xb0, xb1, sem0, sem1):
    wid = lax.axis_index("s") * 2 + lax.axis_index("c")
    base = wid * CHUNK
    pltpu.sync_copy(pos_hbm.at[pl.ds(base, CHUNK)], pos_vm)
    bufs = (xb0, xb1)
    sems = (sem0, sem1)
    cps = [None, None]
    for k2 in range(CHUNK // L):
        b = k2 % 2
        if cps[b] is not None:
            cps[b].wait()
        tok0 = lax.rem(base + k2 * L, T)
        pltpu.sync_copy(xn_hbm.at[pl.ds(tok0, L)], bufs[b])
        pv = pos_vm[pl.ds(k2 * L, L)]
        cps[b] = pltpu.async_copy(bufs[b], xs_hbm.at[pv], sems[b])
    for cp in cps:
        cp.wait()


def _ffn_body(meta_ref, xs_ref, w1_ref, b1_ref, w2_ref, b2_ref, ys_ref):
    i = pl.program_id(0)

    @pl.when(i < meta_ref[NTF])
    def _():
        e = meta_ref[i]
        xv = _unpack_bf16(xs_ref[...]).astype(jnp.bfloat16)
        h = lax.dot_general(xv, w1_ref[0].astype(jnp.bfloat16),
                            (((1,), (1,)), ((), ())),
                            preferred_element_type=jnp.float32)
        h = jnp.maximum(h + b1_ref[pl.ds(e, 1), :], 0.0)
        y = lax.dot_general(h.astype(jnp.bfloat16),
                            w2_ref[0].astype(jnp.bfloat16),
                            (((1,), (1,)), ((), ())),
                            preferred_element_type=jnp.float32)
        ys_ref[...] = _pack_bf16(y + b2_ref[pl.ds(e, 1), :])


_ffn_call = pl.pallas_call(
    _ffn_body,
    grid_spec=pltpu.PrefetchScalarGridSpec(
        num_scalar_prefetch=1,
        grid=(NTF,),
        in_specs=[
            pl.BlockSpec((BM, DP), lambda i, m: (m[64 + i], 0)),
            pl.BlockSpec((1, H, D), lambda i, m: (m[i], 0, 0)),
            pl.BlockSpec((E, H), lambda i, m: (0, 0)),
            pl.BlockSpec((1, D, H), lambda i, m: (m[i], 0, 0)),
            pl.BlockSpec((E, D), lambda i, m: (0, 0)),
        ],
        out_specs=pl.BlockSpec((BM, DP), lambda i, m: (m[64 + i], 0)),
    ),
    out_shape=jax.ShapeDtypeStruct((P, DP), jnp.int32),
    compiler_params=pltpu.CompilerParams(
        dimension_semantics=("arbitrary",)),
)


@functools.partial(
    pl.kernel,
    out_type=[
        jax.ShapeDtypeStruct((T, DP), jnp.int32),
        jax.ShapeDtypeStruct((T, DP), jnp.int32),
    ],
    mesh=_sc_mesh,
    scratch_types=[
        pltpu.VMEM((T // NW,), jnp.int32),
        pltpu.VMEM((T // NW,), jnp.int32),
        pltpu.VMEM((L, DP), jnp.int32),
        pltpu.VMEM((L, DP), jnp.int32),
        pltpu.VMEM((L, DP), jnp.int32),
        pltpu.VMEM((L, DP), jnp.int32),
        pltpu.SemaphoreType.DMA,
        pltpu.SemaphoreType.DMA,
    ],
)
def _gat(pos_hbm, ys_hbm, g0_hbm, g1_hbm,
         p0, p1, ya0, yb0, ya1, yb1, sem0, sem1):
    wid = lax.axis_index("s") * 2 + lax.axis_index("c")
    tpw = T // NW
    t0 = wid * tpw
    pltpu.sync_copy(pos_hbm.at[pl.ds(t0, tpw)], p0)
    pltpu.sync_copy(pos_hbm.at[pl.ds(T + t0, tpw)], p1)
    bufs = ((ya0, yb0), (ya1, yb1))
    sems = (sem0, sem1)
    wcps = [None, None]
    for cc in range(tpw // L):
        b = cc % 2
        y0, y1 = bufs[b]
        if wcps[b] is not None:
            wcps[b][0].wait()
            wcps[b][1].wait()
        tc0 = t0 + cc * L
        i0 = p0[pl.ds(cc * L, L)]
        i1 = p1[pl.ds(cc * L, L)]
        cp0 = pltpu.async_copy(ys_hbm.at[i0], y0, sems[b])
        cp1 = pltpu.async_copy(ys_hbm.at[i1], y1, sems[b])
        cp0.wait()
        cp1.wait()
        wcps[b] = (
            pltpu.async_copy(y0, g0_hbm.at[pl.ds(tc0, L)], sems[b]),
            pltpu.async_copy(y1, g1_hbm.at[pl.ds(tc0, L)], sems[b]),
        )
    for wcp in wcps:
        wcp[0].wait()
        wcp[1].wait()


def _fin_body(x_ref, ww_ref, g0_ref, g1_ref, out_ref):
    sel0 = jnp.where(
        lax.broadcasted_iota(jnp.int32, (E, 8), 0) == 0, 1.0, 0.0)[:, 0:1]
    sel1 = jnp.where(
        lax.broadcasted_iota(jnp.int32, (E, 8), 0) == 1, 1.0, 0.0)[:, 0:1]
    w0c = lax.dot_general(ww_ref[...], sel0, (((0,), (0,)), ((), ())),
                          preferred_element_type=jnp.float32)
    w1c = lax.dot_general(ww_ref[...], sel1, (((0,), (0,)), ((), ())),
                          preferred_element_type=jnp.float32)
    g0 = _unpack_bf16(g0_ref[...])
    g1 = _unpack_bf16(g1_ref[...])
    out_ref[...] = x_ref[...] + w0c * g0 + w1c * g1


_fin_call = pl.pallas_call(
    _fin_body,
    grid=(NT,),
    in_specs=[
        pl.BlockSpec((BT, D), lambda t: (t, 0)),
        pl.BlockSpec((E, BT), lambda t: (0, t)),
        pl.BlockSpec((BT, DP), lambda t: (t, 0)),
        pl.BlockSpec((BT, DP), lambda t: (t, 0)),
    ],
    out_specs=pl.BlockSpec((BT, D), lambda t: (t, 0)),
    out_shape=jax.ShapeDtypeStruct((T, D), jnp.float32),
)


def kernel(x, router_W, router_b, W1, b1, W2, b2, ln_g, ln_b):
    rb2 = jnp.broadcast_to(router_b[:, None], (E, 8))
    g2 = jnp.broadcast_to(ln_g[None, :], (8, D))
    lb2 = jnp.broadcast_to(ln_b[None, :], (8, D))
    xn, ww, pos3, meta8 = _prert_call(x, router_W, rb2, g2, lb2)
    pos = jnp.reshape(pos3, (SLOTS,))
    meta = meta8[0]
    xs = _disp(xn, pos)
    ys = _ffn_call(meta, xs, W1, b1, W2, b2)
    g0, g1 = _gat(pos, ys)
    return _fin_call(x, ww, g0, g1)


# speedup vs baseline: 1.1639x; 1.0077x over previous
import functools

import jax
import jax.numpy as jnp
from jax import lax
from jax.experimental import pallas as pl
from jax.experimental.pallas import tpu as pltpu
from jax.experimental.pallas import tpu_sc as plsc

E = 8
D = 1024
H = 2048
T = 2048
BT = 256
NT = T // BT
BM = 256
SLOTS = 2 * T
P = SLOTS + E * BM
NTF = P // BM
NMETA = 48
TS = 1024
NS = SLOTS // TS

NW = 32
CHUNK = SLOTS // NW
L = 16
DP = D // 2


def _pack_bf16(x):
    u = lax.bitcast_convert_type(x, jnp.uint32)
    b = (u + jnp.uint32(0x7FFF) + ((u >> 16) & jnp.uint32(1))) >> 16
    hi, lo = b[:, :DP], b[:, DP:]
    return lax.bitcast_convert_type((hi << 16) | lo, jnp.int32)


def _unpack_bf16(p):
    pu = lax.bitcast_convert_type(p, jnp.uint32)
    first = lax.bitcast_convert_type(pu & jnp.uint32(0xFFFF0000), jnp.float32)
    second = lax.bitcast_convert_type(pu << 16, jnp.float32)
    return jnp.concatenate([first, second], axis=1)


def _prert_body(x_ref, rw_ref, rb_ref, g_ref, b_ref,
                xn_ref, ww_ref, pos_ref, meta_ref,
                e_scr, rank_scr, acc_scr, tri_scr):
    i = pl.program_id(0)

    @pl.when(i == 0)
    def _():
        acc_scr[...] = jnp.zeros_like(acc_scr)
        ir = lax.broadcasted_iota(jnp.int32, (TS, TS), 0)
        ic = lax.broadcasted_iota(jnp.int32, (TS, TS), 1)
        tri_scr[...] = jnp.where(ir < ic, 1.0, 0.0)

    @pl.when(i < NT)
    def _():
        x = x_ref[...]
        mu = jnp.mean(x, axis=1, keepdims=True)
        xc = x - mu
        var = jnp.mean(xc * xc, axis=1, keepdims=True)
        xn = xc * lax.rsqrt(var + 1e-5) * g_ref[0:1, :] + b_ref[0:1, :]
        xn_ref[...] = _pack_bf16(xn)
        lt = lax.dot_general(rw_ref[...], xn, (((1,), (1,)), ((), ())),
                             preferred_element_type=jnp.float32) + rb_ref[:, 0:1]
        rows = lax.broadcasted_iota(jnp.int32, (E, BT), 0)
        m1 = jnp.max(lt, axis=0, keepdims=True)
        i1 = jnp.min(jnp.where(lt == m1, rows, E), axis=0, keepdims=True)
        l2 = jnp.where(rows == i1, -jnp.inf, lt)
        m2 = jnp.max(l2, axis=0, keepdims=True)
        i2 = jnp.min(jnp.where(l2 == m2, rows, E), axis=0, keepdims=True)
        w1 = jax.nn.sigmoid(m1 - m2)
        ww_ref[...] = jnp.where(rows == 0, w1,
                                jnp.where(rows == 1, 1.0 - w1, 0.0))
        tpb = TS // BT
        col = pl.multiple_of(lax.rem(i, tpb) * BT, BT)
        r0 = i // tpb
        e_scr[pl.ds(r0, 1), pl.ds(col, BT)] = i1
        e_scr[pl.ds(NS // 2 + r0, 1), pl.ds(col, BT)] = i2

    @pl.when((i >= NT) & (i < NT + NS))
    def _():
        t = i - NT
        ev = e_scr[pl.ds(t, 1), :]
        erows = lax.broadcasted_iota(jnp.int32, (E, TS), 0)
        oh = jnp.where(ev == erows, 1.0, 0.0)
        run = acc_scr[:, 0:1]
        within = lax.dot_general(oh, tri_scr[...], (((1,), (0,)), ((), ())),
                                 preferred_element_type=jnp.float32)
        rank = jnp.sum(oh * (within + run), axis=0, keepdims=True)
        rank_scr[pl.ds(t, 1), :] = rank
        newrun = run + jnp.sum(oh, axis=1, keepdims=True)
        acc_scr[:, 0:1] = newrun

        @pl.when(t == NS - 1)
        def _():
            seg = jnp.floor((newrun + (BM - 1)) / BM) * BM
            er = lax.broadcasted_iota(jnp.int32, (E, E), 0)
            ec = lax.broadcasted_iota(jnp.int32, (E, E), 1)
            tri8 = jnp.where(ec < er, 1.0, 0.0)
            off = lax.dot_general(tri8, seg, (((1,), (0,)), ((), ())),
                                  preferred_element_type=jnp.float32)
            acc_scr[:, 1:2] = off
            acc_scr[:, 2:3] = jnp.broadcast_to(
                jnp.sum(seg, axis=0, keepdims=True), (E, 1))

    @pl.when(i >= NT + NS)
    def _():
        t = i - (NT + NS)
        ev = e_scr[pl.ds(t, 1), :]
        erows = lax.broadcasted_iota(jnp.int32, (E, TS), 0)
        oh = jnp.where(ev == erows, 1.0, 0.0)
        off = acc_scr[:, 1:2]
        off_sel = jnp.sum(oh * off, axis=0, keepdims=True)
        pos = off_sel + rank_scr[pl.ds(t, 1), :]
        pos_ref[0] = pos.astype(jnp.int32)

        @pl.when(t == 0)
        def _():
            ivec = lax.broadcasted_iota(jnp.int32, (1, 128), 1)
            rowe = lax.broadcasted_iota(jnp.int32, (E, 128), 0)
            cmp = jnp.where((rowe >= 1)
                            & (ivec.astype(jnp.float32) * BM >= off),
                            1.0, 0.0)
            te = jnp.sum(cmp, axis=0, keepdims=True)
            used = acc_scr[0:1, 2:3] / BM
            te = jnp.where(ivec >= NTF, used, te)
            te = jnp.where(ivec >= 64,
                           jnp.clip(ivec.astype(jnp.float32) - 64.0,
                                    0.0, used - 1.0), te)
            meta_ref[...] = jnp.broadcast_to(te.astype(jnp.int32), (8, 128))


def _clip7(i):
    return jnp.minimum(i, NT - 1)


_prert_call = pl.pallas_call(
    _prert_body,
    grid=(NT + 2 * NS,),
    in_specs=[
        pl.BlockSpec((BT, D), lambda i: (_clip7(i), 0)),
        pl.BlockSpec((E, D), lambda i: (0, 0)),
        pl.BlockSpec((E, 8), lambda i: (0, 0)),
        pl.BlockSpec((8, D), lambda i: (0, 0)),
        pl.BlockSpec((8, D), lambda i: (0, 0)),
    ],
    out_specs=[
        pl.BlockSpec((BT, DP), lambda i: (_clip7(i), 0)),
        pl.BlockSpec((E, BT), lambda i: (0, _clip7(i))),
        pl.BlockSpec((1, 1, TS),
                     lambda i: (jnp.clip(i - (NT + NS), 0, NS - 1), 0, 0)),
        pl.BlockSpec((8, 128), lambda i: (0, 0)),
    ],
    out_shape=[
        jax.ShapeDtypeStruct((T, DP), jnp.int32),
        jax.ShapeDtypeStruct((E, T), jnp.float32),
        jax.ShapeDtypeStruct((NS, 1, TS), jnp.int32),
        jax.ShapeDtypeStruct((8, 128), jnp.int32),
    ],
    scratch_shapes=[
        pltpu.VMEM((NS, TS), jnp.int32),
        pltpu.VMEM((NS, TS), jnp.float32),
        pltpu.VMEM((E, 128), jnp.float32),
        pltpu.VMEM((TS, TS), jnp.float32),
    ],
    compiler_params=pltpu.CompilerParams(
        dimension_semantics=("arbitrary",)),
)


_sc_mesh = plsc.VectorSubcoreMesh(core_axis_name="c", subcore_axis_name="s")


@functools.partial(
    pl.kernel,
    out_type=jax.ShapeDtypeStruct((P, DP), jnp.int32),
    mesh=_sc_mesh,
    scratch_types=[
        pltpu.VMEM((CHUNK,), jnp.int32),
        pltpu.VMEM((L, DP), jnp.int32),
        pltpu.VMEM((L, DP), jnp.int32),
        pltpu.SemaphoreType.DMA,
        pltpu.SemaphoreType.DMA,
    ],
)
def _disp(xn_hbm, pos_hbm, xs_hbm, pos_vm, xb0, xb1, sem0, sem1):
    wid = lax.axis_index("s") * 2 + lax.axis_index("c")
    base = wid * CHUNK
    pltpu.sync_copy(pos_hbm.at[pl.ds(base, CHUNK)], pos_vm)
    bufs = (xb0, xb1)
    sems = (sem0, sem1)
    cps = [None, None]
    for k2 in range(CHUNK // L):
        b = k2 % 2
        if cps[b] is not None:
            cps[b].wait()
        tok0 = lax.rem(base + k2 * L, T)
        pltpu.sync_copy(xn_hbm.at[pl.ds(tok0, L)], bufs[b])
        pv = pos_vm[pl.ds(k2 * L, L)]
        cps[b] = pltpu.async_copy(bufs[b], xs_hbm.at[pv], sems[b])
    for cp in cps:
        cp.wait()


def _ffn_body(meta_ref, xs_ref, w1_ref, b1_ref, w2_ref, b2_ref, ys_ref):
    i = pl.program_id(0)

    @pl.when(i < meta_ref[NTF])
    def _():
        e = meta_ref[i]
        xv = _unpack_bf16(xs_ref[...]).astype(jnp.bfloat16)
        h = lax.dot_general(xv, w1_ref[0].astype(jnp.bfloat16),
                            (((1,), (1,)), ((), ())),
                            preferred_element_type=jnp.float32)
        h = jnp.maximum(h + b1_ref[pl.ds(e, 1), :], 0.0)
        y = lax.dot_general(h.astype(jnp.bfloat16),
                            w2_ref[0].astype(jnp.bfloat16),
                            (((1,), (1,)), ((), ())),
                            preferred_element_type=jnp.float32)
        ys_ref[...] = _pack_bf16(y + b2_ref[pl.ds(e, 1), :])


_ffn_call = pl.pallas_call(
    _ffn_body,
    grid_spec=pltpu.PrefetchScalarGridSpec(
        num_scalar_prefetch=1,
        grid=(NTF,),
        in_specs=[
            pl.BlockSpec((BM, DP), lambda i, m: (m[64 + i], 0)),
            pl.BlockSpec((1, H, D), lambda i, m: (m[i], 0, 0)),
            pl.BlockSpec((E, H), lambda i, m: (0, 0)),
            pl.BlockSpec((1, D, H), lambda i, m: (m[i], 0, 0)),
            pl.BlockSpec((E, D), lambda i, m: (0, 0)),
        ],
        out_specs=pl.BlockSpec((BM, DP), lambda i, m: (m[64 + i], 0)),
    ),
    out_shape=jax.ShapeDtypeStruct((P, DP), jnp.int32),
    compiler_params=pltpu.CompilerParams(
        dimension_semantics=("arbitrary",)),
)


@functools.partial(
    pl.kernel,
    out_type=[
        jax.ShapeDtypeStruct((T, DP), jnp.int32),
        jax.ShapeDtypeStruct((T, DP), jnp.int32),
    ],
    mesh=_sc_mesh,
    scratch_types=[
        pltpu.VMEM((T // NW,), jnp.int32),
        pltpu.VMEM((T // NW,), jnp.int32),
        pltpu.VMEM((L, DP), jnp.int32),
        pltpu.VMEM((L, DP), jnp.int32),
        pltpu.VMEM((L, DP), jnp.int32),
        pltpu.VMEM((L, DP), jnp.int32),
        pltpu.SemaphoreType.DMA,
        pltpu.SemaphoreType.DMA,
    ],
)
def _gat(pos_hbm, ys_hbm, g0_hbm, g1_hbm,
         p0, p1, ya0, yb0, ya1, yb1, sem0, sem1):
    wid = lax.axis_index("s") * 2 + lax.axis_index("c")
    tpw = T // NW
    t0 = wid * tpw
    pltpu.sync_copy(pos_hbm.at[pl.ds(t0, tpw)], p0)
    pltpu.sync_copy(pos_hbm.at[pl.ds(T + t0, tpw)], p1)
    bufs = ((ya0, yb0), (ya1, yb1))
    sems = (sem0, sem1)
    wcps = [None, None]
    for cc in range(tpw // L):
        b = cc % 2
        y0, y1 = bufs[b]
        if wcps[b] is not None:
            wcps[b][0].wait()
            wcps[b][1].wait()
        tc0 = t0 + cc * L
        i0 = p0[pl.ds(cc * L, L)]
        i1 = p1[pl.ds(cc * L, L)]
        cp0 = pltpu.async_copy(ys_hbm.at[i0], y0, sems[b])
        cp1 = pltpu.async_copy(ys_hbm.at[i1], y1, sems[b])
        cp0.wait()
        cp1.wait()
        wcps[b] = (
            pltpu.async_copy(y0, g0_hbm.at[pl.ds(tc0, L)], sems[b]),
            pltpu.async_copy(y1, g1_hbm.at[pl.ds(tc0, L)], sems[b]),
        )
    for wcp in wcps:
        wcp[0].wait()
        wcp[1].wait()


def _fin_body(x_ref, ww_ref, g0_ref, g1_ref, out_ref):
    sel0 = jnp.where(
        lax.broadcasted_iota(jnp.int32, (E, 8), 0) == 0, 1.0, 0.0)[:, 0:1]
    sel1 = jnp.where(
        lax.broadcasted_iota(jnp.int32, (E, 8), 0) == 1, 1.0, 0.0)[:, 0:1]
    w0c = lax.dot_general(ww_ref[...], sel0, (((0,), (0,)), ((), ())),
                          preferred_element_type=jnp.float32)
    w1c = lax.dot_general(ww_ref[...], sel1, (((0,), (0,)), ((), ())),
                          preferred_element_type=jnp.float32)
    g0 = _unpack_bf16(g0_ref[...])
    g1 = _unpack_bf16(g1_ref[...])
    out_ref[...] = x_ref[...] + w0c * g0 + w1c * g1


_fin_call = pl.pallas_call(
    _fin_body,
    grid=(NT,),
    in_specs=[
        pl.BlockSpec((BT, D), lambda t: (t, 0)),
        pl.BlockSpec((E, BT), lambda t: (0, t)),
        pl.BlockSpec((BT, DP), lambda t: (t, 0)),
        pl.BlockSpec((BT, DP), lambda t: (t, 0)),
    ],
    out_specs=pl.BlockSpec((BT, D), lambda t: (t, 0)),
    out_shape=jax.ShapeDtypeStruct((T, D), jnp.float32),
)


def kernel(x, router_W, router_b, W1, b1, W2, b2, ln_g, ln_b):
    rb2 = jnp.broadcast_to(router_b[:, None], (E, 8))
    g2 = jnp.broadcast_to(ln_g[None, :], (8, D))
    lb2 = jnp.broadcast_to(ln_b[None, :], (8, D))
    xn, ww, pos3, meta8 = _prert_call(x, router_W, rb2, g2, lb2)
    pos = jnp.reshape(pos3, (SLOTS,))
    meta = meta8[0]
    xs = _disp(xn, pos)
    ys = _ffn_call(meta, xs, W1, b1, W2, b2)
    g0, g1 = _gat(pos, ys)
    return _fin_call(x, ww, g0, g1)
